# baseline (speedup 1.0000x reference)
import jax
import jax.numpy as jnp
from jax.experimental import pallas as pl


def _route(logits):
    c = logits.shape[-1]
    iota = jax.lax.broadcasted_iota(jnp.int32, logits.shape, 1)
    m1 = jnp.max(logits, axis=-1, keepdims=True)
    idx1 = jnp.min(jnp.where(logits == m1, iota, c), axis=-1, keepdims=True)
    hit1 = iota == idx1
    masked = jnp.where(hit1, -jnp.inf, logits)
    m2 = jnp.max(masked, axis=-1, keepdims=True)
    idx2 = jnp.min(jnp.where(masked == m2, iota, c), axis=-1, keepdims=True)
    hit2 = iota == idx2
    e2 = jnp.exp(m2 - m1)
    p1 = 1.0 / (1.0 + e2)
    p2 = 1.0 - p1
    return jnp.where(hit1, p1, jnp.where(hit2, p2, 0.0))


def _half(x_ref, w1_ref, b1_ref, tb_ref, w2_ref, b2_ref):
    h = jnp.dot(x_ref[0, 0], w1_ref[...], preferred_element_type=jnp.float32)
    h = h + b1_ref[...]
    h = 0.5 * h * (1.0 + jax.lax.erf(h * 0.7071067811865476)) + tb_ref[...]
    logits = jnp.dot(h, w2_ref[...], preferred_element_type=jnp.float32)
    return _route(logits + b2_ref[...])


def _router_tile(xa_ref, xb_ref, w1_ref, b1_ref, tb_ref, w2_ref, b2_ref,
                 out_ref):
    out_ref[0, 0] = _half(xa_ref, w1_ref, b1_ref, tb_ref, w2_ref, b2_ref)
    out_ref[0, 1] = _half(xb_ref, w1_ref, b1_ref, tb_ref, w2_ref, b2_ref)


def kernel(x, W1, b1, W2, b2, task_table, task_id):
    original_shape = x.shape
    xf = x.reshape(-1, x.shape[-1])
    n, d = xf.shape
    e = W1.shape[1]
    c = W2.shape[1]
    tb = task_table[task_id].reshape(1, e)

    tm = 1024
    g = n // (2 * tm)
    x4 = xf.reshape(g, 2, tm, d)
    rep = lambda i: (0, 0)
    probs = pl.pallas_call(
        _router_tile,
        grid=(g,),
        in_specs=[
            pl.BlockSpec((1, 1, tm, d), lambda i: (i, 0, 0, 0)),
            pl.BlockSpec((1, 1, tm, d), lambda i: (i, 1, 0, 0)),
            pl.BlockSpec((d, e), rep),
            pl.BlockSpec((1, e), rep),
            pl.BlockSpec((1, e), rep),
            pl.BlockSpec((e, c), rep),
            pl.BlockSpec((1, c), rep),
        ],
        out_specs=pl.BlockSpec((1, 2, tm, c), lambda i: (i, 0, 0, 0)),
        out_shape=jax.ShapeDtypeStruct((g, 2, tm, c), jnp.float32),
    )(x4, x4, W1, b1.reshape(1, e), tb, W2, b2.reshape(1, c))
    return probs.reshape(*original_shape[:-1], c)
